# baseline (speedup 1.0000x reference)
import functools

import jax
import jax.numpy as jnp
from jax import lax
from jax.experimental import pallas as pl
from jax.experimental.pallas import tpu as pltpu
from jax.experimental.pallas import tpu_sc as plsc

EMB = 64
B, T = 4096, 200
NC, NS = 2, 16
NW = NC * NS
SLAB = B // NW
NG = SLAB // 16
NBUF = 2


def _zindex16(v):
    x = v & 0x3FF
    y = lax.shift_right_logical(v, 10)

    def spread(b):
        b = (b | (b << 8)) & 16711935
        b = (b | (b << 4)) & 252645135
        b = (b | (b << 2)) & 858993459
        b = (b | (b << 1)) & 1431655765
        return b

    return (spread(y) << 1) | spread(x)


_MESH = plsc.VectorSubcoreMesh(core_axis_name="c", subcore_axis_name="s")


@functools.partial(
    pl.kernel,
    out_type=jax.ShapeDtypeStruct((T, EMB, B), jnp.float32),
    mesh=_MESH,
    compiler_params=pltpu.CompilerParams(use_tc_tiling_on_sc=True,
                                         needs_layout_passes=False),
    scratch_types=(
        [pltpu.VMEM((SLAB,), jnp.int32) for _ in range(NBUF)]
        + [pltpu.VMEM((SLAB,), jnp.int32) for _ in range(NBUF)]
        + [pltpu.VMEM((SLAB, 2 * EMB), jnp.float32) for _ in range(NBUF)]
        + [pltpu.VMEM((EMB, SLAB), jnp.float32) for _ in range(NBUF)]
        + [pltpu.SemaphoreType.DMA for _ in range(3 * NBUF)]
    ),
)
def _sc_lookup(loc_t_hbm, table_hbm, out_hbm, *bufs):
    idx = bufs[0:NBUF]
    par = bufs[NBUF:2 * NBUF]
    rows = bufs[2 * NBUF:3 * NBUF]
    tr = bufs[3 * NBUF:4 * NBUF]
    sem_i = bufs[4 * NBUF:5 * NBUF]
    sem_g = bufs[5 * NBUF:6 * NBUF]
    sem_o = bufs[6 * NBUF:7 * NBUF]
    wid = lax.axis_index("s") * NC + lax.axis_index("c")
    slab = wid * SLAB
    iota = lax.iota(jnp.int32, 16)

    def fire_ids(t, b):
        pltpu.async_copy(loc_t_hbm.at[t].at[pl.ds(slab, SLAB)], idx[b],
                         sem_i[b])

    def wait_ids(t, b):
        pltpu.make_async_copy(loc_t_hbm.at[t].at[pl.ds(slab, SLAB)], idx[b],
                              sem_i[b]).wait()

    def zcompute(b):
        for g in range(NG):
            sl = pl.ds(g * 16, 16)
            z = _zindex16(idx[b][sl])
            idx[b][sl] = lax.shift_right_logical(z, 1)
            par[b][sl] = (z & 1) * EMB

    def fire_gather(b):
        pltpu.async_copy(table_hbm.at[idx[b]], rows[b], sem_g[b])

    def wait_gather(b):
        pltpu.make_async_copy(table_hbm.at[idx[b]], rows[b], sem_g[b]).wait()

    def transpose(b):
        for g in range(NG):
            bidx = iota + (g * 16)
            parv = par[b][pl.ds(g * 16, 16)]
            for e in range(EMB):
                v = plsc.load_gather(rows[b], [bidx, parv + e])
                tr[b][e, pl.ds(g * 16, 16)] = v

    def fire_out(t, b):
        pltpu.async_copy(tr[b], out_hbm.at[t].at[:, pl.ds(slab, SLAB)],
                         sem_o[b])

    def wait_out(t, b):
        pltpu.make_async_copy(tr[b], out_hbm.at[t].at[:, pl.ds(slab, SLAB)],
                              sem_o[b]).wait()

    for k in range(NBUF):
        fire_ids(k, k)
    wait_ids(0, 0)
    zcompute(0)
    fire_gather(0)

    def step(t, carry):
        for b in range(NBUF):

            @pl.when(t % NBUF == b)
            def _(b=b):
                nb = (b + 1) % NBUF

                @pl.when(t + 1 < T)
                def _():
                    wait_ids(t + 1, nb)
                    zcompute(nb)
                    fire_gather(nb)

                wait_gather(b)

                @pl.when(t + NBUF < T)
                def _():
                    fire_ids(t + NBUF, b)

                @pl.when(t >= NBUF)
                def _():
                    wait_out(t - NBUF, b)

                transpose(b)
                fire_out(t, b)

        return carry

    lax.fori_loop(0, T, step, 0)

    for k in range(NBUF):
        t = T - NBUF + k
        wait_out(t, t % NBUF)


def kernel(location_id, table):
    loc_t = location_id.T
    table_pairs = table.reshape(table.shape[0] // 2, 2 * table.shape[1])
    out_t = _sc_lookup(loc_t, table_pairs)
    return jnp.transpose(out_t, (2, 0, 1))


# speedup vs baseline: 1.0059x; 1.0059x over previous
import functools

import jax
import jax.numpy as jnp
from jax import lax
from jax.experimental import pallas as pl
from jax.experimental.pallas import tpu as pltpu
from jax.experimental.pallas import tpu_sc as plsc

EMB = 64
B, T = 4096, 200
NC, NS = 2, 16
NW = NC * NS
SLAB = B // NW
NG = SLAB // 16
NBUF = 4


def _zindex16(v):
    x = v & 0x3FF
    y = lax.shift_right_logical(v, 10)

    def spread(b):
        b = (b | (b << 8)) & 16711935
        b = (b | (b << 4)) & 252645135
        b = (b | (b << 2)) & 858993459
        b = (b | (b << 1)) & 1431655765
        return b

    return (spread(y) << 1) | spread(x)


_MESH = plsc.VectorSubcoreMesh(core_axis_name="c", subcore_axis_name="s")


@functools.partial(
    pl.kernel,
    out_type=jax.ShapeDtypeStruct((T, EMB, B), jnp.float32),
    mesh=_MESH,
    compiler_params=pltpu.CompilerParams(use_tc_tiling_on_sc=True,
                                         needs_layout_passes=False),
    scratch_types=(
        [pltpu.VMEM((SLAB,), jnp.int32) for _ in range(NBUF)]
        + [pltpu.VMEM((SLAB,), jnp.int32) for _ in range(NBUF)]
        + [pltpu.VMEM((SLAB, 2 * EMB), jnp.float32) for _ in range(NBUF)]
        + [pltpu.VMEM((EMB, SLAB), jnp.float32) for _ in range(NBUF)]
        + [pltpu.SemaphoreType.DMA for _ in range(3 * NBUF)]
    ),
)
def _sc_lookup(loc_t_hbm, table_hbm, out_hbm, *bufs):
    idx = bufs[0:NBUF]
    par = bufs[NBUF:2 * NBUF]
    rows = bufs[2 * NBUF:3 * NBUF]
    tr = bufs[3 * NBUF:4 * NBUF]
    sem_i = bufs[4 * NBUF:5 * NBUF]
    sem_g = bufs[5 * NBUF:6 * NBUF]
    sem_o = bufs[6 * NBUF:7 * NBUF]
    wid = lax.axis_index("s") * NC + lax.axis_index("c")
    slab = wid * SLAB
    iota = lax.iota(jnp.int32, 16)

    def fire_ids(t, b):
        pltpu.async_copy(loc_t_hbm.at[t].at[pl.ds(slab, SLAB)], idx[b],
                         sem_i[b])

    def wait_ids(t, b):
        pltpu.make_async_copy(loc_t_hbm.at[t].at[pl.ds(slab, SLAB)], idx[b],
                              sem_i[b]).wait()

    def zcompute(b):
        for g in range(NG):
            sl = pl.ds(g * 16, 16)
            z = _zindex16(idx[b][sl])
            idx[b][sl] = lax.shift_right_logical(z, 1)
            par[b][sl] = (z & 1) * EMB

    def fire_gather(b):
        pltpu.async_copy(table_hbm.at[idx[b]], rows[b], sem_g[b])

    def wait_gather(b):
        pltpu.make_async_copy(table_hbm.at[idx[b]], rows[b], sem_g[b]).wait()

    def transpose(b):
        def tstep(g, carry):
            bidx = iota + g * 16
            parv = par[b][pl.ds(g * 16, 16)]
            for e in range(EMB):
                v = plsc.load_gather(rows[b], [bidx, parv + e])
                tr[b][e, pl.ds(g * 16, 16)] = v
            return carry

        lax.fori_loop(0, NG, tstep, 0)

    def fire_out(t, b):
        pltpu.async_copy(tr[b], out_hbm.at[t].at[:, pl.ds(slab, SLAB)],
                         sem_o[b])

    def wait_out(t, b):
        pltpu.make_async_copy(tr[b], out_hbm.at[t].at[:, pl.ds(slab, SLAB)],
                              sem_o[b]).wait()

    for k in range(NBUF):
        fire_ids(k, k)
    wait_ids(0, 0)
    zcompute(0)
    fire_gather(0)

    def step(t, carry):
        for b in range(NBUF):

            @pl.when(t % NBUF == b)
            def _(b=b):
                nb = (b + 1) % NBUF

                @pl.when(t + 1 < T)
                def _():
                    wait_ids(t + 1, nb)
                    zcompute(nb)
                    fire_gather(nb)

                wait_gather(b)

                @pl.when(t + NBUF < T)
                def _():
                    fire_ids(t + NBUF, b)

                @pl.when(t >= NBUF)
                def _():
                    wait_out(t - NBUF, b)

                transpose(b)
                fire_out(t, b)

        return carry

    lax.fori_loop(0, T, step, 0)

    for k in range(NBUF):
        t = T - NBUF + k
        wait_out(t, t % NBUF)


def kernel(location_id, table):
    loc_t = location_id.T
    table_pairs = table.reshape(table.shape[0] // 2, 2 * table.shape[1])
    out_t = _sc_lookup(loc_t, table_pairs)
    return jnp.transpose(out_t, (2, 0, 1))


# speedup vs baseline: 1.4893x; 1.4805x over previous
import functools

import jax
import jax.numpy as jnp
from jax import lax
from jax.experimental import pallas as pl
from jax.experimental.pallas import tpu as pltpu
from jax.experimental.pallas import tpu_sc as plsc

EMB = 64
N = 4096 * 200
NC, NS = 2, 16
NW = NC * NS
PER_W = N // NW
CH = 512
NCHUNK = PER_W // CH
IPG = 128
GPC = CH // IPG
NBUF = 3


def _zindex16(v):
    x = v & 0x3FF
    y = lax.shift_right_logical(v, 10)

    def spread(b):
        b = (b | (b << 8)) & 16711935
        b = (b | (b << 4)) & 252645135
        b = (b | (b << 2)) & 858993459
        b = (b | (b << 1)) & 1431655765
        return b

    return (spread(y) << 1) | spread(x)


_MESH = plsc.VectorSubcoreMesh(core_axis_name="c", subcore_axis_name="s")


@functools.partial(
    pl.kernel,
    out_type=jax.ShapeDtypeStruct((N, EMB), jnp.float32),
    mesh=_MESH,
    compiler_params=pltpu.CompilerParams(use_tc_tiling_on_sc=False),
    scratch_types=[
        pltpu.VMEM((PER_W,), jnp.int32),
        pltpu.VMEM((NBUF, CH, EMB), jnp.float32),
        pltpu.SemaphoreType.DMA,
        pltpu.SemaphoreType.DMA,
        pltpu.SemaphoreType.DMA,
        pltpu.SemaphoreType.DMA,
        pltpu.SemaphoreType.DMA,
        pltpu.SemaphoreType.DMA,
    ],
)
def _sc_lookup(loc_hbm, table_hbm, out_hbm, idx_all, rows, sg0, sg1, sg2,
               so0, so1, so2):
    sem_g = (sg0, sg1, sg2)
    sem_o = (so0, so1, so2)
    wid = lax.axis_index("s") * NC + lax.axis_index("c")
    base = wid * PER_W

    pltpu.sync_copy(loc_hbm.at[pl.ds(base, PER_W)], idx_all)

    def zstep(i, carry):
        sl = pl.ds(i * 16, 16)
        idx_all[sl] = _zindex16(idx_all[sl])
        return carry

    lax.fori_loop(0, PER_W // 16, zstep, 0)

    def fire_gathers(c, b):
        for j in range(GPC):
            pltpu.async_copy(
                table_hbm.at[idx_all.at[pl.ds(c * CH + j * IPG, IPG)]],
                rows.at[b].at[pl.ds(j * IPG, IPG)],
                sem_g[b],
            )

    def wait_gathers(c, b):
        for j in range(GPC):
            pltpu.make_async_copy(
                table_hbm.at[idx_all.at[pl.ds(c * CH + j * IPG, IPG)]],
                rows.at[b].at[pl.ds(j * IPG, IPG)],
                sem_g[b],
            ).wait()

    def fire_out(c, b):
        pltpu.async_copy(rows.at[b], out_hbm.at[pl.ds(base + c * CH, CH)],
                         sem_o[b])

    def wait_out(c, b):
        pltpu.make_async_copy(rows.at[b], out_hbm.at[pl.ds(base + c * CH, CH)],
                              sem_o[b]).wait()

    for k in range(NBUF):
        fire_gathers(k, k)

    def step(c, carry):
        @pl.when(jnp.logical_and(c > 0, c + NBUF - 1 < NCHUNK))
        def _refill():
            for b in range(NBUF):

                @pl.when((c - 1) % NBUF == b)
                def _():
                    wait_out(c - 1, b)
                    fire_gathers(c + NBUF - 1, b)

        for b in range(NBUF):

            @pl.when(c % NBUF == b)
            def _drain():
                wait_gathers(c, b)
                fire_out(c, b)

        return carry

    lax.fori_loop(0, NCHUNK, step, 0)

    for k in range(NBUF):
        c = NCHUNK - NBUF + k
        wait_out(c, c % NBUF)


def kernel(location_id, table):
    flat = location_id.reshape(-1)
    out = _sc_lookup(flat, table)
    return out.reshape(location_id.shape + (EMB,))
